# baseline (speedup 1.0000x reference)
import functools

import jax
import jax.numpy as jnp
from jax import lax
from jax.experimental import pallas as pl
from jax.experimental.pallas import tpu as pltpu
from jax.experimental.pallas import tpu_sc as plsc

_NC, _NS = 2, 16
_NW = _NC * _NS

N_TOK = 4096
HIDDEN = 2048
FF = 1024
N_ROUTED = 8
TOTAL_EXPERTS = 10
TOP_K = 2
SCALE = 2.5

TILE = 256
A = N_TOK * TOP_K
NT = (A + N_ROUTED * (TILE - 1) + TILE - 1) // TILE
P = NT * TILE
RB = 512


def _sc_mesh():
    return plsc.VectorSubcoreMesh(core_axis_name="c", subcore_axis_name="s",
                                  num_cores=_NC, num_subcores=_NS)


def _sc_gather(table, idx):
    n_rows = idx.shape[0]
    d = table.shape[1]
    rows_per_w = n_rows // _NW
    ch = min(32, rows_per_w)
    n_ch = rows_per_w // ch

    def body(table_hbm, idx_hbm, out_hbm, idx_v, rows_v, sem):
        wid = lax.axis_index("s") * _NC + lax.axis_index("c")
        base = wid * rows_per_w

        def step(c, carry):
            off = base + c * ch
            pltpu.sync_copy(idx_hbm.at[pl.ds(off, ch)], idx_v)
            pltpu.async_copy(table_hbm.at[idx_v], rows_v, sem).wait()
            pltpu.sync_copy(rows_v, out_hbm.at[pl.ds(off, ch)])
            return carry

        lax.fori_loop(0, n_ch, step, 0)

    f = pl.kernel(
        body,
        out_type=jax.ShapeDtypeStruct((n_rows, d), table.dtype),
        mesh=_sc_mesh(),
        scratch_types=[
            pltpu.VMEM((ch,), jnp.int32),
            pltpu.VMEM((ch, d), table.dtype),
            pltpu.SemaphoreType.DMA,
        ],
    )
    return f(table, idx)


def _sc_combine(idwh, ogw, p0, p1):
    rows_per_w = N_TOK // _NW
    ch = 16
    n_ch = rows_per_w // ch
    hb = HIDDEN // 16

    def body(idwh_hbm, ogw_hbm, p0_hbm, p1_hbm, out_hbm,
             i0v, i1v, r0v, r1v, hv, s0, s1, s2):
        wid = lax.axis_index("s") * _NC + lax.axis_index("c")
        base = wid * rows_per_w

        def step(c, carry):
            off = base + c * ch
            pltpu.sync_copy(p0_hbm.at[pl.ds(off, ch)], i0v)
            pltpu.sync_copy(p1_hbm.at[pl.ds(off, ch)], i1v)
            cp0 = pltpu.async_copy(ogw_hbm.at[i0v], r0v, s0)
            cp1 = pltpu.async_copy(ogw_hbm.at[i1v], r1v, s1)
            cp2 = pltpu.async_copy(idwh_hbm.at[pl.ds(off, ch)], hv, s2)
            cp0.wait()
            cp1.wait()
            cp2.wait()

            def row(i, cr):
                def col(j, cr2):
                    sl = pl.ds(j * 16, 16)
                    hv[i, sl] = hv[i, sl] + r0v[i, sl] + r1v[i, sl]
                    return cr2

                return lax.fori_loop(0, hb, col, cr, unroll=8)

            lax.fori_loop(0, ch, row, 0)
            pltpu.sync_copy(hv, out_hbm.at[pl.ds(off, ch)])
            return carry

        lax.fori_loop(0, n_ch, step, 0)

    f = pl.kernel(
        body,
        out_type=jax.ShapeDtypeStruct((N_TOK, HIDDEN), jnp.float32),
        mesh=_sc_mesh(),
        scratch_types=[
            pltpu.VMEM((ch,), jnp.int32),
            pltpu.VMEM((ch,), jnp.int32),
            pltpu.VMEM((ch, HIDDEN), jnp.float32),
            pltpu.VMEM((ch, HIDDEN), jnp.float32),
            pltpu.VMEM((ch, HIDDEN), jnp.float32),
            pltpu.SemaphoreType.DMA,
            pltpu.SemaphoreType.DMA,
            pltpu.SemaphoreType.DMA,
        ],
    )
    return f(idwh, ogw, p0, p1)


def _router_kernel(h_ref, wc_ref, bias_ref, sel_ref, w_ref, idwh_ref):
    h = h_ref[...]
    logits = lax.dot_general(h, wc_ref[...], (((1,), (1,)), ((), ())),
                             preferred_element_type=jnp.float32)
    m = jnp.max(logits, axis=-1, keepdims=True)
    e = jnp.exp(logits - m)
    probs = e / jnp.sum(e, axis=-1, keepdims=True)
    biased = probs + bias_ref[...]
    iota = lax.broadcasted_iota(jnp.int32, biased.shape, 1)
    big = jnp.full_like(iota, TOTAL_EXPERTS)
    v1 = jnp.max(biased, axis=-1, keepdims=True)
    i1 = jnp.min(jnp.where(biased == v1, iota, big), axis=-1, keepdims=True)
    b2 = jnp.where(iota == i1, -jnp.inf, biased)
    v2 = jnp.max(b2, axis=-1, keepdims=True)
    i2 = jnp.min(jnp.where(b2 == v2, iota, big), axis=-1, keepdims=True)
    w1 = jnp.sum(jnp.where(iota == i1, probs, 0.0), axis=-1, keepdims=True) * SCALE
    w2 = jnp.sum(jnp.where(iota == i2, probs, 0.0), axis=-1, keepdims=True) * SCALE
    sel_ref[...] = jnp.concatenate([i1, i2], axis=1)
    w_ref[...] = jnp.concatenate([w1, w2], axis=1)
    id_w = w1 * (i1 >= N_ROUTED) + w2 * (i2 >= N_ROUTED)
    idwh_ref[...] = h * id_w


def _router(h, wc, bias, interpret=False):
    grid = (N_TOK // RB,)
    return pl.pallas_call(
        _router_kernel,
        grid=grid,
        in_specs=[
            pl.BlockSpec((RB, HIDDEN), lambda i: (i, 0)),
            pl.BlockSpec((TOTAL_EXPERTS, HIDDEN), lambda i: (0, 0)),
            pl.BlockSpec((1, TOTAL_EXPERTS), lambda i: (0, 0)),
        ],
        out_specs=[
            pl.BlockSpec((RB, TOP_K), lambda i: (i, 0)),
            pl.BlockSpec((RB, TOP_K), lambda i: (i, 0)),
            pl.BlockSpec((RB, HIDDEN), lambda i: (i, 0)),
        ],
        out_shape=[
            jax.ShapeDtypeStruct((N_TOK, TOP_K), jnp.int32),
            jax.ShapeDtypeStruct((N_TOK, TOP_K), jnp.float32),
            jax.ShapeDtypeStruct((N_TOK, HIDDEN), jnp.float32),
        ],
        interpret=interpret,
    )(h, wc, bias.reshape(1, TOTAL_EXPERTS))


def _mlp_kernel(te_ref, hg_ref, gw_ref, uw_ref, dw_ref, rw_ref, out_ref):
    hg = hg_ref[...].astype(jnp.bfloat16)
    g = lax.dot_general(hg, gw_ref[0], (((1,), (1,)), ((), ())),
                        preferred_element_type=jnp.float32)
    u = lax.dot_general(hg, uw_ref[0], (((1,), (1,)), ((), ())),
                        preferred_element_type=jnp.float32)
    inter = (g * jax.nn.sigmoid(g)) * u * rw_ref[0]
    out_ref[...] = lax.dot_general(inter.astype(jnp.bfloat16), dw_ref[0],
                                   (((1,), (1,)), ((), ())),
                                   preferred_element_type=jnp.float32)


def _grouped_mlp(hg, gw, uw, dw, row_w, tile_expert, interpret=False):
    rw3 = row_w.reshape(NT, TILE, 1)
    grid_spec = pltpu.PrefetchScalarGridSpec(
        num_scalar_prefetch=1,
        grid=(NT,),
        in_specs=[
            pl.BlockSpec((TILE, HIDDEN), lambda i, te: (i, 0)),
            pl.BlockSpec((1, FF, HIDDEN), lambda i, te: (te[i], 0, 0)),
            pl.BlockSpec((1, FF, HIDDEN), lambda i, te: (te[i], 0, 0)),
            pl.BlockSpec((1, HIDDEN, FF), lambda i, te: (te[i], 0, 0)),
            pl.BlockSpec((1, TILE, 1), lambda i, te: (i, 0, 0)),
        ],
        out_specs=pl.BlockSpec((TILE, HIDDEN), lambda i, te: (i, 0)),
    )
    return pl.pallas_call(
        _mlp_kernel,
        grid_spec=grid_spec,
        out_shape=jax.ShapeDtypeStruct((P, HIDDEN), jnp.float32),
        compiler_params=pltpu.CompilerParams(
            dimension_semantics=("arbitrary",)),
        interpret=interpret,
    )(tile_expert, hg, gw, uw, dw, rw3)


def _combine_kernel(idwh_ref, g0_ref, g1_ref, out_ref):
    out_ref[...] = idwh_ref[...] + g0_ref[...] + g1_ref[...]


def _combine(idwh, g0, g1, interpret=False):
    grid = (N_TOK // RB,)
    bs = pl.BlockSpec((RB, HIDDEN), lambda i: (i, 0))
    return pl.pallas_call(
        _combine_kernel,
        grid=grid,
        in_specs=[bs, bs, bs],
        out_specs=bs,
        out_shape=jax.ShapeDtypeStruct((N_TOK, HIDDEN), jnp.float32),
        interpret=interpret,
    )(idwh, g0, g1)


def _dispatch_indices(sel, w):
    eid = sel.reshape(-1)
    wf = w.reshape(-1)
    routed = eid < N_ROUTED
    onehot = (eid[:, None] == jnp.arange(N_ROUTED)[None, :]).astype(jnp.int32)
    ranks_all = jnp.cumsum(onehot, axis=0) - onehot
    rank = jnp.sum(ranks_all * onehot, axis=1)
    counts = jnp.sum(onehot, axis=0)
    padded = ((counts + TILE - 1) // TILE) * TILE
    starts = jnp.concatenate([jnp.zeros(1, padded.dtype), jnp.cumsum(padded)[:-1]])
    pos = jnp.where(routed, starts[jnp.clip(eid, 0, N_ROUTED - 1)] + rank,
                    P - 1).astype(jnp.int32)
    a_token = (jnp.arange(A) // TOP_K).astype(jnp.int32)
    token_of_pos = jnp.zeros((P,), jnp.int32).at[pos].set(
        jnp.where(routed, a_token, 0))
    row_w = jnp.zeros((P,), jnp.float32).at[pos].set(jnp.where(routed, wf, 0.0))
    ends = starts + padded
    tile_expert = jnp.minimum(
        jnp.sum((jnp.arange(NT)[:, None] * TILE >= ends[None, :]).astype(jnp.int32),
                axis=1), N_ROUTED - 1).astype(jnp.int32)
    return pos, token_of_pos, row_w, tile_expert


def kernel(hidden_states, classifier_w, e_score_correction_bias,
           gate_w, up_w, down_w, interpret=False):
    sel, w, idwh = _router(hidden_states, classifier_w,
                           e_score_correction_bias, interpret=interpret)
    pos, token_of_pos, row_w, tile_expert = _dispatch_indices(sel, w)
    if interpret:
        hg = jnp.take(hidden_states, token_of_pos, axis=0)
    else:
        hg = _sc_gather(hidden_states, token_of_pos)
    ogw = _grouped_mlp(hg, gate_w.astype(jnp.bfloat16),
                       up_w.astype(jnp.bfloat16), down_w.astype(jnp.bfloat16),
                       row_w, tile_expert, interpret=interpret)
    p01 = pos.reshape(N_TOK, TOP_K)
    if interpret:
        g0 = jnp.take(ogw, p01[:, 0], axis=0)
        g1 = jnp.take(ogw, p01[:, 1], axis=0)
        return _combine(idwh, g0, g1, interpret=interpret)
    return _sc_combine(idwh, ogw, p01[:, 0], p01[:, 1])


# speedup vs baseline: 1.0737x; 1.0737x over previous
import functools

import jax
import jax.numpy as jnp
from jax import lax
from jax.experimental import pallas as pl
from jax.experimental.pallas import tpu as pltpu
from jax.experimental.pallas import tpu_sc as plsc

_NC, _NS = 2, 16
_NW = _NC * _NS

N_TOK = 4096
HIDDEN = 2048
FF = 1024
N_ROUTED = 8
TOTAL_EXPERTS = 10
TOP_K = 2
SCALE = 2.5

TILE = 128
A = N_TOK * TOP_K
NT = (A + N_ROUTED * (TILE - 1) + TILE - 1) // TILE
P = NT * TILE
RB = 512


def _sc_mesh():
    return plsc.VectorSubcoreMesh(core_axis_name="c", subcore_axis_name="s",
                                  num_cores=_NC, num_subcores=_NS)


_GCH = 24
_CCH = 8


def _sc_gather(table, idx3):
    nw, n_ch, ch = idx3.shape
    d = table.shape[1]
    rows_per_w = n_ch * ch
    idx2 = idx3.reshape(nw * rows_per_w)

    def body(table_hbm, idx_hbm, out_hbm, i0, i1, b0, b1,
             si0, si1, g0, g1, w0, w1):
        wid = lax.axis_index("s") * _NC + lax.axis_index("c")
        base = wid * rows_per_w
        idxs, bufs = (i0, i1), (b0, b1)
        sis, gs, ws = (si0, si1), (g0, g1), (w0, w1)
        cp_i = [None, None]
        cp_g = [None, None]
        cp_w = [None, None]
        cp_i[0] = pltpu.async_copy(idx_hbm.at[pl.ds(base, ch)], idxs[0],
                                   sis[0])
        cp_i[0].wait()
        cp_g[0] = pltpu.async_copy(table_hbm.at[i0], bufs[0], gs[0])
        for c in range(n_ch):
            b = c & 1
            nb = (c + 1) & 1
            if c + 1 < n_ch:
                if cp_w[nb] is not None:
                    cp_w[nb].wait()
                cp_i[nb] = pltpu.async_copy(
                    idx_hbm.at[pl.ds(base + (c + 1) * ch, ch)],
                    idxs[nb], sis[nb])
            cp_g[b].wait()
            cp_w[b] = pltpu.async_copy(
                bufs[b], out_hbm.at[pl.ds(base + c * ch, ch)], ws[b])
            if c + 1 < n_ch:
                cp_i[nb].wait()
                cp_g[nb] = pltpu.async_copy(
                    table_hbm.at[idxs[nb]], bufs[nb], gs[nb])
        cp_w[(n_ch - 1) & 1].wait()
        if n_ch > 1:
            cp_w[n_ch & 1].wait()

    f = pl.kernel(
        body,
        out_type=jax.ShapeDtypeStruct((nw * rows_per_w, d), table.dtype),
        mesh=_sc_mesh(),
        scratch_types=[
            pltpu.VMEM((ch,), jnp.int32),
            pltpu.VMEM((ch,), jnp.int32),
            pltpu.VMEM((ch, d), table.dtype),
            pltpu.VMEM((ch, d), table.dtype),
            pltpu.SemaphoreType.DMA,
            pltpu.SemaphoreType.DMA,
            pltpu.SemaphoreType.DMA,
            pltpu.SemaphoreType.DMA,
            pltpu.SemaphoreType.DMA,
            pltpu.SemaphoreType.DMA,
        ],
    )
    return f(table, idx2)


def _sc_combine(idwh, ogw, pos_flat):
    ch = _CCH
    rows_per_w = N_TOK // _NW
    n_ch = rows_per_w // ch
    hp = HIDDEN // 2
    hl = hp // 16

    def body(idwh_hbm, ogw_hbm, pos_hbm, out_hbm,
             i0, i1, r0, r1, hv0, hv1,
             si0, si1, sg0, sg1, sl0, sl1, sw0, sw1):
        wid = lax.axis_index("s") * _NC + lax.axis_index("c")
        base = wid * rows_per_w
        idxs, rs, hvs = (i0, i1), (r0, r1), (hv0, hv1)
        sis, sgs, sls, sws = (si0, si1), (sg0, sg1), (sl0, sl1), (sw0, sw1)
        cp_i = [None, None]
        cp_g = [None, None]
        cp_l = [None, None]
        cp_w = [None, None]

        def issue_front(c, s):
            cp_i[s] = pltpu.async_copy(
                pos_hbm.at[pl.ds((base + c * ch) * 2, 2 * ch)],
                idxs[s], sis[s])
            cp_l[s] = pltpu.async_copy(
                idwh_hbm.at[pl.ds(base + c * ch, ch)], hvs[s], sls[s])

        def issue_gather(s):
            cp_i[s].wait()
            cp_g[s] = pltpu.async_copy(ogw_hbm.at[idxs[s]], rs[s], sgs[s])

        issue_front(0, 0)
        issue_gather(0)
        for c in range(n_ch):
            b = c & 1
            nb = (c + 1) & 1
            if c + 1 < n_ch:
                if cp_w[nb] is not None:
                    cp_w[nb].wait()
                issue_front(c + 1, nb)
                issue_gather(nb)
            cp_g[b].wait()
            cp_l[b].wait()
            hv, rv = hvs[b], rs[b]

            def add_chunk(j, carry):
                i = j >> 6
                k16 = (j & 63) * 16
                sl = pl.ds(k16, 16)
                r0 = rv[2 * i, sl]
                r1 = rv[2 * i + 1, sl]
                a0 = lax.bitcast_convert_type(
                    lax.shift_left(r0, 16), jnp.float32)
                b0 = lax.bitcast_convert_type(
                    r0 & jnp.int32(-65536), jnp.float32)
                a1 = lax.bitcast_convert_type(
                    lax.shift_left(r1, 16), jnp.float32)
                b1 = lax.bitcast_convert_type(
                    r1 & jnp.int32(-65536), jnp.float32)
                hv[i, sl] = hv[i, sl] + a0 + a1
                slh = pl.ds(hp + k16, 16)
                hv[i, slh] = hv[i, slh] + b0 + b1
                return carry

            lax.fori_loop(0, ch * hl, add_chunk, 0, unroll=8)
            cp_w[b] = pltpu.async_copy(
                hv, out_hbm.at[pl.ds(base + c * ch, ch)], sws[b])
        cp_w[(n_ch - 1) & 1].wait()
        if n_ch > 1:
            cp_w[n_ch & 1].wait()

    f = pl.kernel(
        body,
        out_type=jax.ShapeDtypeStruct((N_TOK, HIDDEN), jnp.float32),
        mesh=_sc_mesh(),
        scratch_types=[
            pltpu.VMEM((2 * ch,), jnp.int32),
            pltpu.VMEM((2 * ch,), jnp.int32),
            pltpu.VMEM((2 * ch, HIDDEN // 2), jnp.int32),
            pltpu.VMEM((2 * ch, HIDDEN // 2), jnp.int32),
            pltpu.VMEM((ch, HIDDEN), jnp.float32),
            pltpu.VMEM((ch, HIDDEN), jnp.float32),
            pltpu.SemaphoreType.DMA,
            pltpu.SemaphoreType.DMA,
            pltpu.SemaphoreType.DMA,
            pltpu.SemaphoreType.DMA,
            pltpu.SemaphoreType.DMA,
            pltpu.SemaphoreType.DMA,
            pltpu.SemaphoreType.DMA,
            pltpu.SemaphoreType.DMA,
        ],
    )
    return f(idwh, ogw, pos_flat)


def _pack_bf16(x_lo, x_hi):
    u_lo = lax.bitcast_convert_type(x_lo, jnp.int32) + 0x8000
    u_hi = lax.bitcast_convert_type(x_hi, jnp.int32) + 0x8000
    return (lax.shift_right_logical(u_lo, 16) |
            (u_hi & jnp.int32(-65536)))


def _unpack_bf16(p):
    lo = lax.bitcast_convert_type(lax.shift_left(p, 16), jnp.float32)
    hi = lax.bitcast_convert_type(p & jnp.int32(-65536), jnp.float32)
    return lo, hi


def _router_kernel(h_ref, wc_ref, bias_ref, sel_ref, w_ref, idwh_ref,
                   hp_ref):
    h = h_ref[...]
    logits = lax.dot_general(h, wc_ref[...], (((1,), (1,)), ((), ())),
                             preferred_element_type=jnp.float32)
    m = jnp.max(logits, axis=-1, keepdims=True)
    e = jnp.exp(logits - m)
    probs = e / jnp.sum(e, axis=-1, keepdims=True)
    biased = probs + bias_ref[...]
    iota = lax.broadcasted_iota(jnp.int32, biased.shape, 1)
    big = jnp.full_like(iota, TOTAL_EXPERTS)
    v1 = jnp.max(biased, axis=-1, keepdims=True)
    i1 = jnp.min(jnp.where(biased == v1, iota, big), axis=-1, keepdims=True)
    b2 = jnp.where(iota == i1, -jnp.inf, biased)
    v2 = jnp.max(b2, axis=-1, keepdims=True)
    i2 = jnp.min(jnp.where(b2 == v2, iota, big), axis=-1, keepdims=True)
    w1 = jnp.sum(jnp.where(iota == i1, probs, 0.0), axis=-1, keepdims=True) * SCALE
    w2 = jnp.sum(jnp.where(iota == i2, probs, 0.0), axis=-1, keepdims=True) * SCALE
    sel_ref[...] = jnp.concatenate([i1, i2], axis=1)
    w_ref[...] = jnp.concatenate([w1, w2], axis=1)
    id_w = w1 * (i1 >= N_ROUTED) + w2 * (i2 >= N_ROUTED)
    idwh_ref[...] = h * id_w
    hp_ref[...] = _pack_bf16(h[:, :HIDDEN // 2], h[:, HIDDEN // 2:])


def _router(h, wc, bias, interpret=False):
    grid = (N_TOK // RB,)
    return pl.pallas_call(
        _router_kernel,
        grid=grid,
        in_specs=[
            pl.BlockSpec((RB, HIDDEN), lambda i: (i, 0)),
            pl.BlockSpec((TOTAL_EXPERTS, HIDDEN), lambda i: (0, 0)),
            pl.BlockSpec((1, TOTAL_EXPERTS), lambda i: (0, 0)),
        ],
        out_specs=[
            pl.BlockSpec((RB, TOP_K), lambda i: (i, 0)),
            pl.BlockSpec((RB, TOP_K), lambda i: (i, 0)),
            pl.BlockSpec((RB, HIDDEN), lambda i: (i, 0)),
            pl.BlockSpec((RB, HIDDEN // 2), lambda i: (i, 0)),
        ],
        out_shape=[
            jax.ShapeDtypeStruct((N_TOK, TOP_K), jnp.int32),
            jax.ShapeDtypeStruct((N_TOK, TOP_K), jnp.float32),
            jax.ShapeDtypeStruct((N_TOK, HIDDEN), jnp.float32),
            jax.ShapeDtypeStruct((N_TOK, HIDDEN // 2), jnp.int32),
        ],
        interpret=interpret,
    )(h, wc, bias.reshape(1, TOTAL_EXPERTS))


def _mlp_kernel(te_ref, hg_ref, gw_ref, uw_ref, dw_ref, rw_ref, out_ref):
    lo, hi = _unpack_bf16(hg_ref[...])
    hg = jnp.concatenate([lo, hi], axis=1).astype(jnp.bfloat16)
    g = lax.dot_general(hg, gw_ref[0].astype(jnp.bfloat16),
                        (((1,), (1,)), ((), ())),
                        preferred_element_type=jnp.float32)
    u = lax.dot_general(hg, uw_ref[0].astype(jnp.bfloat16),
                        (((1,), (1,)), ((), ())),
                        preferred_element_type=jnp.float32)
    inter = (g * jax.nn.sigmoid(g)) * u * rw_ref[0]
    o = lax.dot_general(inter.astype(jnp.bfloat16),
                        dw_ref[0].astype(jnp.bfloat16),
                        (((1,), (1,)), ((), ())),
                        preferred_element_type=jnp.float32)
    out_ref[...] = _pack_bf16(o[:, :HIDDEN // 2], o[:, HIDDEN // 2:])


def _grouped_mlp(hg, gw, uw, dw, row_w, tile_expert, interpret=False):
    rw3 = row_w.reshape(NT, TILE, 1)
    grid_spec = pltpu.PrefetchScalarGridSpec(
        num_scalar_prefetch=1,
        grid=(NT,),
        in_specs=[
            pl.BlockSpec((TILE, HIDDEN // 2), lambda i, te: (i, 0)),
            pl.BlockSpec((1, FF, HIDDEN), lambda i, te: (te[i], 0, 0)),
            pl.BlockSpec((1, FF, HIDDEN), lambda i, te: (te[i], 0, 0)),
            pl.BlockSpec((1, HIDDEN, FF), lambda i, te: (te[i], 0, 0)),
            pl.BlockSpec((1, TILE, 1), lambda i, te: (i, 0, 0)),
        ],
        out_specs=pl.BlockSpec((TILE, HIDDEN // 2), lambda i, te: (i, 0)),
    )
    return pl.pallas_call(
        _mlp_kernel,
        grid_spec=grid_spec,
        out_shape=jax.ShapeDtypeStruct((P, HIDDEN // 2), jnp.int32),
        compiler_params=pltpu.CompilerParams(
            dimension_semantics=("arbitrary",)),
        interpret=interpret,
    )(tile_expert, hg, gw, uw, dw, rw3)


def _combine_kernel(idwh_ref, g0_ref, g1_ref, out_ref):
    out_ref[...] = idwh_ref[...] + g0_ref[...] + g1_ref[...]


def _combine(idwh, g0, g1, interpret=False):
    grid = (N_TOK // RB,)
    bs = pl.BlockSpec((RB, HIDDEN), lambda i: (i, 0))
    return pl.pallas_call(
        _combine_kernel,
        grid=grid,
        in_specs=[bs, bs, bs],
        out_specs=bs,
        out_shape=jax.ShapeDtypeStruct((N_TOK, HIDDEN), jnp.float32),
        interpret=interpret,
    )(idwh, g0, g1)


def _dispatch_indices(sel, w):
    eid = sel.reshape(-1)
    wf = w.reshape(-1)
    routed = eid < N_ROUTED
    onehot = (eid[:, None] == jnp.arange(N_ROUTED)[None, :]).astype(jnp.int32)
    ranks_all = jnp.cumsum(onehot, axis=0) - onehot
    rank = jnp.sum(ranks_all * onehot, axis=1)
    counts = jnp.sum(onehot, axis=0)
    padded = ((counts + TILE - 1) // TILE) * TILE
    starts = jnp.concatenate([jnp.zeros(1, padded.dtype), jnp.cumsum(padded)[:-1]])
    pos = jnp.where(routed, starts[jnp.clip(eid, 0, N_ROUTED - 1)] + rank,
                    P - 1).astype(jnp.int32)
    a_token = (jnp.arange(A) // TOP_K).astype(jnp.int32)
    token_of_pos = jnp.zeros((P,), jnp.int32).at[pos].set(
        jnp.where(routed, a_token, 0))
    row_w = jnp.zeros((P,), jnp.float32).at[pos].set(jnp.where(routed, wf, 0.0))
    ends = starts + padded
    tile_expert = jnp.minimum(
        jnp.sum((jnp.arange(NT)[:, None] * TILE >= ends[None, :]).astype(jnp.int32),
                axis=1), N_ROUTED - 1).astype(jnp.int32)
    return pos, token_of_pos, row_w, tile_expert


def kernel(hidden_states, classifier_w, e_score_correction_bias,
           gate_w, up_w, down_w, interpret=False):
    sel, w, idwh, hp32 = _router(hidden_states, classifier_w,
                                 e_score_correction_bias, interpret=interpret)
    pos, token_of_pos, row_w, tile_expert = _dispatch_indices(sel, w)
    if interpret:
        hg32 = jnp.take(hp32, token_of_pos, axis=0)
    else:
        hg32 = _sc_gather(hp32,
                          token_of_pos.reshape(_NW, P // (_NW * _GCH), _GCH))
    ogw = _grouped_mlp(hg32, gate_w, up_w, down_w,
                       row_w, tile_expert, interpret=interpret)
    if interpret:
        lo = lax.bitcast_convert_type(lax.shift_left(ogw, 16), jnp.float32)
        hi = lax.bitcast_convert_type(ogw & jnp.int32(-65536), jnp.float32)
        og = jnp.concatenate([lo, hi], axis=1)
        p01 = pos.reshape(N_TOK, TOP_K)
        g0 = jnp.take(og, p01[:, 0], axis=0)
        g1 = jnp.take(og, p01[:, 1], axis=0)
        return _combine(idwh, g0, g1, interpret=interpret)
    return _sc_combine(idwh, ogw, pos)


# speedup vs baseline: 1.2776x; 1.1899x over previous
import jax
import jax.numpy as jnp
from jax import lax
from jax.experimental import pallas as pl
from jax.experimental.pallas import tpu as pltpu
from jax.experimental.pallas import tpu_sc as plsc

_NC, _NS = 2, 16
_NW = _NC * _NS

N_TOK = 4096
HIDDEN = 2048
FF = 1024
N_ROUTED = 8
TOTAL_EXPERTS = 10
TOP_K = 2
SCALE = 2.5

TILE = 256
A = N_TOK * TOP_K
NT = (A + N_ROUTED * (TILE - 1) + TILE - 1) // TILE
P = NT * TILE
RB = 512


def _sc_mesh():
    return plsc.VectorSubcoreMesh(core_axis_name="c", subcore_axis_name="s",
                                  num_cores=_NC, num_subcores=_NS)


_GCH = 40
_CCH = 8


def _sc_gather(table, idx3):
    nw, n_ch, ch = idx3.shape
    d = table.shape[1]
    rows_per_w = n_ch * ch
    idx2 = idx3.reshape(nw * rows_per_w)

    def body(table_hbm, idx_hbm, out_hbm, i0, i1, b0, b1,
             si0, si1, g0, g1, w0, w1):
        wid = lax.axis_index("s") * _NC + lax.axis_index("c")
        base = wid * rows_per_w
        idxs, bufs = (i0, i1), (b0, b1)
        sis, gs, ws = (si0, si1), (g0, g1), (w0, w1)
        cp_i = [None, None]
        cp_g = [None, None]
        cp_w = [None, None]
        cp_i[0] = pltpu.async_copy(idx_hbm.at[pl.ds(base, ch)], idxs[0],
                                   sis[0])
        cp_i[0].wait()
        cp_g[0] = pltpu.async_copy(table_hbm.at[i0], bufs[0], gs[0])
        for c in range(n_ch):
            b = c & 1
            nb = (c + 1) & 1
            if c + 1 < n_ch:
                if cp_w[nb] is not None:
                    cp_w[nb].wait()
                cp_i[nb] = pltpu.async_copy(
                    idx_hbm.at[pl.ds(base + (c + 1) * ch, ch)],
                    idxs[nb], sis[nb])
            cp_g[b].wait()
            cp_w[b] = pltpu.async_copy(
                bufs[b], out_hbm.at[pl.ds(base + c * ch, ch)], ws[b])
            if c + 1 < n_ch:
                cp_i[nb].wait()
                cp_g[nb] = pltpu.async_copy(
                    table_hbm.at[idxs[nb]], bufs[nb], gs[nb])
        cp_w[(n_ch - 1) & 1].wait()
        if n_ch > 1:
            cp_w[n_ch & 1].wait()

    f = pl.kernel(
        body,
        out_type=jax.ShapeDtypeStruct((nw * rows_per_w, d), table.dtype),
        mesh=_sc_mesh(),
        scratch_types=[
            pltpu.VMEM((ch,), jnp.int32),
            pltpu.VMEM((ch,), jnp.int32),
            pltpu.VMEM((ch, d), table.dtype),
            pltpu.VMEM((ch, d), table.dtype),
            pltpu.SemaphoreType.DMA,
            pltpu.SemaphoreType.DMA,
            pltpu.SemaphoreType.DMA,
            pltpu.SemaphoreType.DMA,
            pltpu.SemaphoreType.DMA,
            pltpu.SemaphoreType.DMA,
        ],
    )
    return f(table, idx2)


def _sc_combine(idwh, ogw, pos_flat):
    ch = _CCH
    rows_per_w = N_TOK // _NW
    n_ch = rows_per_w // ch
    hp = HIDDEN // 2
    hl = hp // 16

    def body(idwh_hbm, ogw_hbm, pos_hbm, out_hbm,
             i0, i1, r0, r1, hv0, hv1,
             si0, si1, sg0, sg1, sl0, sl1, sw0, sw1):
        wid = lax.axis_index("s") * _NC + lax.axis_index("c")
        base = wid * rows_per_w
        idxs, rs, hvs = (i0, i1), (r0, r1), (hv0, hv1)
        sis, sgs, sls, sws = (si0, si1), (sg0, sg1), (sl0, sl1), (sw0, sw1)
        cp_i = [None, None]
        cp_g = [None, None]
        cp_l = [None, None]
        cp_w = [None, None]

        def issue_front(c, s):
            cp_i[s] = pltpu.async_copy(
                pos_hbm.at[pl.ds((base + c * ch) * 2, 2 * ch)],
                idxs[s], sis[s])
            cp_l[s] = pltpu.async_copy(
                idwh_hbm.at[pl.ds(base + c * ch, ch)], hvs[s], sls[s])

        def issue_gather(s):
            cp_i[s].wait()
            cp_g[s] = pltpu.async_copy(ogw_hbm.at[idxs[s]], rs[s], sgs[s])

        issue_front(0, 0)
        issue_gather(0)
        for c in range(n_ch):
            b = c & 1
            nb = (c + 1) & 1
            if c + 1 < n_ch:
                if cp_w[nb] is not None:
                    cp_w[nb].wait()
                issue_front(c + 1, nb)
                issue_gather(nb)
            cp_g[b].wait()
            cp_l[b].wait()
            hv, rv = hvs[b], rs[b]

            def add_chunk(j, carry):
                i = j >> 6
                k16 = (j & 63) * 16
                sl = pl.ds(k16, 16)
                r0 = rv[2 * i, sl]
                r1 = rv[2 * i + 1, sl]
                a0 = lax.bitcast_convert_type(
                    lax.shift_left(r0, 16), jnp.float32)
                b0 = lax.bitcast_convert_type(
                    r0 & jnp.int32(-65536), jnp.float32)
                a1 = lax.bitcast_convert_type(
                    lax.shift_left(r1, 16), jnp.float32)
                b1 = lax.bitcast_convert_type(
                    r1 & jnp.int32(-65536), jnp.float32)
                hv[i, sl] = hv[i, sl] + a0 + a1
                slh = pl.ds(hp + k16, 16)
                hv[i, slh] = hv[i, slh] + b0 + b1
                return carry

            lax.fori_loop(0, ch * hl, add_chunk, 0, unroll=8)
            cp_w[b] = pltpu.async_copy(
                hv, out_hbm.at[pl.ds(base + c * ch, ch)], sws[b])
        cp_w[(n_ch - 1) & 1].wait()
        if n_ch > 1:
            cp_w[n_ch & 1].wait()

    f = pl.kernel(
        body,
        out_type=jax.ShapeDtypeStruct((N_TOK, HIDDEN), jnp.float32),
        mesh=_sc_mesh(),
        scratch_types=[
            pltpu.VMEM((2 * ch,), jnp.int32),
            pltpu.VMEM((2 * ch,), jnp.int32),
            pltpu.VMEM((2 * ch, HIDDEN // 2), jnp.int32),
            pltpu.VMEM((2 * ch, HIDDEN // 2), jnp.int32),
            pltpu.VMEM((ch, HIDDEN), jnp.float32),
            pltpu.VMEM((ch, HIDDEN), jnp.float32),
            pltpu.SemaphoreType.DMA,
            pltpu.SemaphoreType.DMA,
            pltpu.SemaphoreType.DMA,
            pltpu.SemaphoreType.DMA,
            pltpu.SemaphoreType.DMA,
            pltpu.SemaphoreType.DMA,
            pltpu.SemaphoreType.DMA,
            pltpu.SemaphoreType.DMA,
        ],
    )
    return f(idwh, ogw, pos_flat)


def _pack_bf16(x_lo, x_hi):
    u_lo = lax.bitcast_convert_type(x_lo, jnp.int32) + 0x8000
    u_hi = lax.bitcast_convert_type(x_hi, jnp.int32) + 0x8000
    return (lax.shift_right_logical(u_lo, 16) |
            (u_hi & jnp.int32(-65536)))


def _unpack_bf16(p):
    lo = lax.bitcast_convert_type(lax.shift_left(p, 16), jnp.float32)
    hi = lax.bitcast_convert_type(p & jnp.int32(-65536), jnp.float32)
    return lo, hi


def _router_kernel(h_ref, wc_ref, bias_ref, sel_ref, w_ref, idwh_ref,
                   hp_ref):
    h = h_ref[...]
    logits = lax.dot_general(h, wc_ref[...], (((1,), (1,)), ((), ())),
                             preferred_element_type=jnp.float32)
    m = jnp.max(logits, axis=-1, keepdims=True)
    e = jnp.exp(logits - m)
    probs = e / jnp.sum(e, axis=-1, keepdims=True)
    biased = probs + bias_ref[...]
    iota = lax.broadcasted_iota(jnp.int32, biased.shape, 1)
    big = jnp.full_like(iota, TOTAL_EXPERTS)
    v1 = jnp.max(biased, axis=-1, keepdims=True)
    i1 = jnp.min(jnp.where(biased == v1, iota, big), axis=-1, keepdims=True)
    b2 = jnp.where(iota == i1, -jnp.inf, biased)
    v2 = jnp.max(b2, axis=-1, keepdims=True)
    i2 = jnp.min(jnp.where(b2 == v2, iota, big), axis=-1, keepdims=True)
    w1 = jnp.sum(jnp.where(iota == i1, probs, 0.0), axis=-1, keepdims=True) * SCALE
    w2 = jnp.sum(jnp.where(iota == i2, probs, 0.0), axis=-1, keepdims=True) * SCALE
    sel_ref[...] = jnp.concatenate([i1, i2], axis=1)
    w_ref[...] = jnp.concatenate([w1, w2], axis=1)
    id_w = w1 * (i1 >= N_ROUTED) + w2 * (i2 >= N_ROUTED)
    idwh_ref[...] = h * id_w
    hp_ref[...] = _pack_bf16(h[:, :HIDDEN // 2], h[:, HIDDEN // 2:])


def _router(h, wc, bias):
    grid = (N_TOK // RB,)
    return pl.pallas_call(
        _router_kernel,
        grid=grid,
        in_specs=[
            pl.BlockSpec((RB, HIDDEN), lambda i: (i, 0)),
            pl.BlockSpec((TOTAL_EXPERTS, HIDDEN), lambda i: (0, 0)),
            pl.BlockSpec((1, TOTAL_EXPERTS), lambda i: (0, 0)),
        ],
        out_specs=[
            pl.BlockSpec((RB, TOP_K), lambda i: (i, 0)),
            pl.BlockSpec((RB, TOP_K), lambda i: (i, 0)),
            pl.BlockSpec((RB, HIDDEN), lambda i: (i, 0)),
            pl.BlockSpec((RB, HIDDEN // 2), lambda i: (i, 0)),
        ],
        out_shape=[
            jax.ShapeDtypeStruct((N_TOK, TOP_K), jnp.int32),
            jax.ShapeDtypeStruct((N_TOK, TOP_K), jnp.float32),
            jax.ShapeDtypeStruct((N_TOK, HIDDEN), jnp.float32),
            jax.ShapeDtypeStruct((N_TOK, HIDDEN // 2), jnp.int32),
        ],
    )(h, wc, bias.reshape(1, TOTAL_EXPERTS))


def _mlp_kernel(te_ref, hg_ref, gw_ref, uw_ref, dw_ref, rw_ref, out_ref):
    lo, hi = _unpack_bf16(hg_ref[...])
    hg = jnp.concatenate([lo, hi], axis=1).astype(jnp.bfloat16)
    g = lax.dot_general(hg, gw_ref[0].astype(jnp.bfloat16),
                        (((1,), (1,)), ((), ())),
                        preferred_element_type=jnp.float32)
    u = lax.dot_general(hg, uw_ref[0].astype(jnp.bfloat16),
                        (((1,), (1,)), ((), ())),
                        preferred_element_type=jnp.float32)
    inter = (g * jax.nn.sigmoid(g)) * u * rw_ref[0]
    o = lax.dot_general(inter.astype(jnp.bfloat16),
                        dw_ref[0].astype(jnp.bfloat16),
                        (((1,), (1,)), ((), ())),
                        preferred_element_type=jnp.float32)
    out_ref[...] = _pack_bf16(o[:, :HIDDEN // 2], o[:, HIDDEN // 2:])


def _grouped_mlp(hg, gw, uw, dw, row_w, tile_expert):
    rw3 = row_w.reshape(NT, TILE, 1)
    grid_spec = pltpu.PrefetchScalarGridSpec(
        num_scalar_prefetch=1,
        grid=(NT,),
        in_specs=[
            pl.BlockSpec((TILE, HIDDEN // 2), lambda i, te: (i, 0)),
            pl.BlockSpec((1, FF, HIDDEN), lambda i, te: (te[i], 0, 0)),
            pl.BlockSpec((1, FF, HIDDEN), lambda i, te: (te[i], 0, 0)),
            pl.BlockSpec((1, HIDDEN, FF), lambda i, te: (te[i], 0, 0)),
            pl.BlockSpec((1, TILE, 1), lambda i, te: (i, 0, 0)),
        ],
        out_specs=pl.BlockSpec((TILE, HIDDEN // 2), lambda i, te: (i, 0)),
    )
    return pl.pallas_call(
        _mlp_kernel,
        grid_spec=grid_spec,
        out_shape=jax.ShapeDtypeStruct((P, HIDDEN // 2), jnp.int32),
        compiler_params=pltpu.CompilerParams(
            dimension_semantics=("arbitrary",)),
    )(tile_expert, hg, gw, uw, dw, rw3)


def _dispatch_indices(sel, w):
    eid = sel.reshape(-1)
    wf = w.reshape(-1)
    routed = eid < N_ROUTED
    onehot = (eid[:, None] == jnp.arange(N_ROUTED)[None, :]).astype(jnp.int32)
    ranks_all = jnp.cumsum(onehot, axis=0) - onehot
    rank = jnp.sum(ranks_all * onehot, axis=1)
    counts = jnp.sum(onehot, axis=0)
    padded = ((counts + TILE - 1) // TILE) * TILE
    starts = jnp.concatenate([jnp.zeros(1, padded.dtype), jnp.cumsum(padded)[:-1]])
    pos = jnp.where(routed, starts[jnp.clip(eid, 0, N_ROUTED - 1)] + rank,
                    P - 1).astype(jnp.int32)
    a_token = (jnp.arange(A) // TOP_K).astype(jnp.int32)
    token_of_pos = jnp.zeros((P,), jnp.int32).at[pos].set(
        jnp.where(routed, a_token, 0))
    row_w = jnp.zeros((P,), jnp.float32).at[pos].set(jnp.where(routed, wf, 0.0))
    ends = starts + padded
    tile_expert = jnp.minimum(
        jnp.sum((jnp.arange(NT)[:, None] * TILE >= ends[None, :]).astype(jnp.int32),
                axis=1), N_ROUTED - 1).astype(jnp.int32)
    return pos, token_of_pos, row_w, tile_expert


def kernel(hidden_states, classifier_w, e_score_correction_bias,
           gate_w, up_w, down_w):
    sel, w, idwh, hp32 = _router(hidden_states, classifier_w,
                                 e_score_correction_bias)
    pos, token_of_pos, row_w, tile_expert = _dispatch_indices(sel, w)
    hg32 = _sc_gather(hp32,
                      token_of_pos.reshape(_NW, P // (_NW * _GCH), _GCH))
    ogw = _grouped_mlp(hg32, gate_w, up_w, down_w, row_w, tile_expert)
    return _sc_combine(idwh, ogw, pos)
